# baseline (speedup 1.0000x reference)
import jax
import jax.numpy as jnp
from jax import lax
from jax.experimental import pallas as pl
from jax.experimental.pallas import tpu as pltpu
from jax.experimental.pallas import tpu_sc as plsc

B, T, G, D = 1024, 50, 4, 128
OUT = 1000
R = 256
NW = 32
BPW = B // NW
L = 16
SHARD_BITS = 14
NSHARD = ((100000 - 1) >> SHARD_BITS) + 1
GROUPS = BPW * R // L
LISTCAP = BPW * R + NSHARD * 128
PAD_META = 255 << 22


def _sc_body(idx_hbm, spans_hbm, table_hbm, sent_hbm,
             idx_v, spans_v, wbuf_v, lmeta_v, lidx_v, rows_v, out_v,
             sem0, sem1):
    cc = lax.axis_index("c")
    ss = lax.axis_index("s")
    wid = ss * 2 + cc
    base = wid * BPW

    pltpu.sync_copy(idx_hbm.at[pl.ds(base, BPW)], idx_v)
    pltpu.sync_copy(spans_hbm.at[pl.ds(base, BPW)], spans_v)

    def weights_batch(j, _):
        for i in range(R // L):
            r = jnp.arange(L, dtype=jnp.int32) + (i * L)
            q = r - 50
            t = jnp.clip(q >> 2, 0, T - 1)
            g = q & 3
            span = plsc.load_gather(spans_v, [jnp.broadcast_to(j, (L,)), t])
            denom = jnp.maximum(span, 1).astype(jnp.float32)
            w_ng = jnp.where(g < span, 1.0 / denom, 0.0)
            w = jnp.where(r < 50, 1.0,
                          jnp.where(r < 250, w_ng, 0.0)) * (1.0 / T)
            wbuf_v[j, pl.ds(i * L, L)] = w
        return ()

    lax.fori_loop(0, BPW, weights_batch, ())

    def zero_out(j, _):
        for d in range(D // L):
            out_v[j, pl.ds(d * L, L)] = jnp.zeros((L,), jnp.float32)
        return ()

    lax.fori_loop(0, BPW, zero_out, ())

    def load_group(g):
        j = g >> 4
        ch = (g >> 3) & 1
        q = (g & 7) * L
        return j, plsc.load_gather(
            idx_v, [jnp.broadcast_to(j, (L,)),
                    jnp.broadcast_to(ch, (L,)),
                    jnp.broadcast_to(q, (L,)) + jnp.arange(L, dtype=jnp.int32)])

    def hist_step(g, cnts):
        _, iv = load_group(g)
        sh = iv >> SHARD_BITS
        return tuple(cnts[k] + jnp.sum((sh == k).astype(jnp.int32))
                     for k in range(NSHARD))

    cnts = lax.fori_loop(0, GROUPS, hist_step, (jnp.int32(0),) * NSHARD)

    offs = []
    acc = jnp.int32(0)
    for k in range(NSHARD):
        offs.append(acc)
        acc = acc + (((cnts[k] + 127) >> 7) << 7)
    total_pad = acc

    def prefill(g, _):
        lmeta_v[pl.ds(g * L, L)] = jnp.full((L,), PAD_META, jnp.int32)
        lidx_v[pl.ds(g * L, L)] = jnp.zeros((L,), jnp.int32)
        return ()

    lax.fori_loop(0, LISTCAP // L, prefill, ())

    def place_step(g, offs):
        j, iv = load_group(g)
        slot = (jnp.broadcast_to((g & 15) * L, (L,))
                + jnp.arange(L, dtype=jnp.int32))
        mv = iv | (j << 17) | (slot << 22)
        sh = iv >> SHARD_BITS
        new = []
        for k in range(NSHARD):
            mk = sh == k
            plsc.store_compressed(lmeta_v.at[pl.ds(offs[k], L)], mv, mask=mk)
            plsc.store_compressed(lidx_v.at[pl.ds(offs[k], L)], iv, mask=mk)
            new.append(offs[k] + jnp.sum(mk.astype(jnp.int32)))
        return tuple(new)

    lax.fori_loop(0, GROUPS, place_step, tuple(offs))

    def issue(jsc, p, sem):
        for cch in range(2):
            pltpu.async_copy(
                table_hbm.at[lidx_v.at[pl.ds(jsc * 256 + cch * 128, 128)]],
                rows_v.at[p, pl.ds(cch * 128, 128)], sem)

    def gwait(p, sem):
        for cch in range(2):
            pltpu.make_async_copy(table_hbm.at[lidx_v.at[pl.ds(0, 128)]],
                                  rows_v.at[p, pl.ds(cch * 128, 128)],
                                  sem).wait()

    def pool(jsc, p):
        col = jnp.arange(L, dtype=jnp.int32)

        def ent_step(r4, _):
            e0 = jsc * 256 + r4 * 4
            ms = [plsc.load_gather(lmeta_v, [jnp.broadcast_to(e0 + u, (L,))])
                  for u in range(4)]
            bvs = [(m >> 17) & 31 for m in ms]
            svs = [m >> 22 for m in ms]
            ws = [plsc.load_gather(wbuf_v, [bvs[u], svs[u]]) for u in range(4)]
            for d in range(D // L):
                for u in range(4):
                    x = ws[u] * rows_v[p, r4 * 4 + u, pl.ds(d * L, L)]
                    plsc.addupdate_scatter(out_v, [bvs[u], col + (d * L)], x)
            return ()

        lax.fori_loop(0, 64, ent_step, ())

    nsc = (total_pad + 255) >> 8
    issue(0, 0, sem0)

    def super_chunk(jsc, _):
        p = jsc & 1

        @pl.when(p == 0)
        def _():
            gwait(0, sem0)

        @pl.when(p == 1)
        def _():
            gwait(1, sem1)

        @pl.when(jsc + 1 < nsc)
        def _():
            @pl.when(p == 0)
            def _():
                issue(jsc + 1, 1, sem1)

            @pl.when(p == 1)
            def _():
                issue(jsc + 1, 0, sem0)

        pool(jsc, p)
        return ()

    lax.fori_loop(0, nsc, super_chunk, ())
    pltpu.sync_copy(out_v, sent_hbm.at[pl.ds(base, BPW)])


def _sc_pool(idx_all, spans_pad, table):
    mesh = plsc.VectorSubcoreMesh(core_axis_name="c", subcore_axis_name="s")
    fn = pl.kernel(
        _sc_body,
        out_type=jax.ShapeDtypeStruct((B, D), jnp.float32),
        mesh=mesh,
        scratch_types=[
            pltpu.VMEM((BPW, 2, 128), jnp.int32),
            pltpu.VMEM((BPW, 64), jnp.int32),
            pltpu.VMEM((BPW, R), jnp.float32),
            pltpu.VMEM((LISTCAP,), jnp.int32),
            pltpu.VMEM((LISTCAP,), jnp.int32),
            pltpu.VMEM((2, 256, D), jnp.float32),
            pltpu.VMEM((BPW, D), jnp.float32),
            pltpu.SemaphoreType.DMA,
            pltpu.SemaphoreType.DMA,
        ],
        compiler_params=pltpu.CompilerParams(needs_layout_passes=False),
    )
    return fn(idx_all, spans_pad, table)


def _mm_body(x_ref, w_ref, b_ref, o_ref):
    o_ref[...] = lax.dot_general(
        x_ref[...], w_ref[...], (((1,), (1,)), ((), ())),
        preferred_element_type=jnp.float32,
        precision=lax.Precision.HIGHEST,
    ) + b_ref[...]


@jax.jit
def _fc(sent, w, b2):
    return pl.pallas_call(
        _mm_body,
        out_shape=jax.ShapeDtypeStruct((B, OUT), jnp.float32),
    )(sent, w, b2)


@jax.jit
def kernel(ngrams, words, word_spans, table, W, b):
    idx_all = jnp.concatenate(
        [words, ngrams.reshape(B, T * G),
         jnp.zeros((B, R - T - T * G), jnp.int32)], axis=1).reshape(B, 2, 128)
    spans_pad = jnp.concatenate(
        [word_spans, jnp.zeros((B, 64 - T), jnp.int32)], axis=1)
    sent = _sc_pool(idx_all, spans_pad, table)
    return _fc(sent, W, b[None, :])


# speedup vs baseline: 3.3703x; 3.3703x over previous
import jax
import jax.numpy as jnp
from jax import lax
from jax.experimental import pallas as pl
from jax.experimental.pallas import tpu as pltpu
from jax.experimental.pallas import tpu_sc as plsc

B, T, G, D = 1024, 50, 4, 128
OUT = 1000
R = 256
NW = 32
BPW = B // NW
L = 16


def _sc_body(idx_hbm, spans_hbm, table_hbm, sent_hbm,
             idx_v, spans_v, wbuf_v, rows0_v, rows1_v, out_v, sem0, sem1):
    c = lax.axis_index("c")
    s = lax.axis_index("s")
    wid = s * 2 + c
    base = wid * BPW

    pltpu.sync_copy(idx_hbm.at[pl.ds(base, BPW)], idx_v)
    pltpu.sync_copy(spans_hbm.at[pl.ds(base, BPW)], spans_v)

    def gather(j, rows, sem):
        for cch in range(2):
            pltpu.async_copy(table_hbm.at[idx_v.at[j, cch]],
                             rows.at[pl.ds(cch * 128, 128)], sem)

    def gwait(rows, sem):
        for cch in range(2):
            pltpu.make_async_copy(table_hbm.at[idx_v.at[0, cch]],
                                  rows.at[pl.ds(cch * 128, 128)], sem).wait()

    gather(0, rows0_v, sem0)

    def weights_batch(j, _):
        for i in range(R // L):
            r = jnp.arange(L, dtype=jnp.int32) + (i * L)
            q = r - 50
            t = jnp.clip(q >> 2, 0, T - 1)
            g = q & 3
            span = plsc.load_gather(spans_v, [jnp.broadcast_to(j, (L,)), t])
            denom = jnp.maximum(span, 1).astype(jnp.float32)
            w_ng = jnp.where(g < span, 1.0 / denom, 0.0)
            w = jnp.where(r < 50, 1.0,
                          jnp.where(r < 250, w_ng, 0.0)) * (1.0 / T)
            wbuf_v[j, pl.ds(i * L, L)] = w
        return ()

    lax.fori_loop(0, BPW, weights_batch, ())

    def pool(j, rows):
        def acc_step(r, a):
            w = plsc.load_gather(
                wbuf_v, [jnp.broadcast_to(j, (L,)), jnp.broadcast_to(r, (L,))])
            return tuple(a[d] + w * rows[r, pl.ds(d * L, L)]
                         for d in range(D // L))
        zeros = (jnp.zeros((L,), jnp.float32),) * (D // L)
        acc = plsc.parallel_loop(0, R, unroll=8, carry=zeros)(acc_step)
        for d in range(D // L):
            out_v[j, pl.ds(d * L, L)] = acc[d]

    def two_batches(k, _):
        j0 = 2 * k
        gwait(rows0_v, sem0)
        gather(j0 + 1, rows1_v, sem1)
        pool(j0, rows0_v)
        gwait(rows1_v, sem1)

        @pl.when(j0 + 2 < BPW)
        def _():
            gather(j0 + 2, rows0_v, sem0)

        pool(j0 + 1, rows1_v)
        return ()

    lax.fori_loop(0, BPW // 2, two_batches, ())
    pltpu.sync_copy(out_v, sent_hbm.at[pl.ds(base, BPW)])


@jax.jit
def _sc_pool(idx_all, spans_pad, table):
    mesh = plsc.VectorSubcoreMesh(core_axis_name="c", subcore_axis_name="s")
    fn = pl.kernel(
        _sc_body,
        out_type=jax.ShapeDtypeStruct((B, D), jnp.float32),
        mesh=mesh,
        scratch_types=[
            pltpu.VMEM((BPW, 2, 128), jnp.int32),
            pltpu.VMEM((BPW, 64), jnp.int32),
            pltpu.VMEM((BPW, R), jnp.float32),
            pltpu.VMEM((R, D), jnp.float32),
            pltpu.VMEM((R, D), jnp.float32),
            pltpu.VMEM((BPW, D), jnp.float32),
            pltpu.SemaphoreType.DMA,
            pltpu.SemaphoreType.DMA,
        ],
        compiler_params=pltpu.CompilerParams(needs_layout_passes=False),
    )
    return fn(idx_all, spans_pad, table)


def _mm_body(x_ref, w_ref, b_ref, o_ref):
    o_ref[...] = lax.dot_general(
        x_ref[...], w_ref[...], (((1,), (1,)), ((), ())),
        preferred_element_type=jnp.float32,
        precision=lax.Precision.HIGHEST,
    ) + b_ref[...]


@jax.jit
def _fc(sent, w, b2):
    return pl.pallas_call(
        _mm_body,
        out_shape=jax.ShapeDtypeStruct((B, OUT), jnp.float32),
    )(sent, w, b2)


@jax.jit
def kernel(ngrams, words, word_spans, table, W, b):
    idx_all = jnp.concatenate(
        [words, ngrams.reshape(B, T * G),
         jnp.zeros((B, R - T - T * G), jnp.int32)], axis=1).reshape(B, 2, 128)
    spans_pad = jnp.concatenate(
        [word_spans, jnp.zeros((B, 64 - T), jnp.int32)], axis=1)
    sent = _sc_pool(idx_all, spans_pad, table)
    return _fc(sent, W, b[None, :])
